# baseline (speedup 1.0000x reference)
import functools

import jax
import jax.numpy as jnp
from jax.experimental import pallas as pl
from jax.experimental.pallas import tpu as pltpu

_K = 8192
_D = 32
_BR = 112
_BK = 512
_BIG_I32 = 2**31 - 1


def _tree_sum(vals):
    n = len(vals)
    while n > 1:
        h = n // 2
        vals = [vals[i] + vals[i + h] for i in range(h)]
        n = h
    return vals[0]


def _vq_kernel(z_ref, c_ref, out_ref, best_val, best_idx):
    k_blk = pl.program_id(1)
    nk = pl.num_programs(1)

    z = z_ref[...]
    c = c_ref[...]

    sq = []
    for d in range(_D):
        t = z[:, d][:, None] - c[:, d][None, :]
        sq.append(t * t)
    dist2 = _tree_sum(sq)

    rowmin = jnp.min(dist2, axis=1)
    kidx = jax.lax.broadcasted_iota(jnp.int32, dist2.shape, 1) + k_blk * _BK
    rowidx = jnp.min(
        jnp.where(dist2 == rowmin[:, None], kidx, jnp.int32(_BIG_I32)), axis=1)

    @pl.when(k_blk == 0)
    def _init():
        best_val[...] = rowmin
        best_idx[...] = rowidx

    @pl.when(k_blk != 0)
    def _update():
        bv = best_val[...]
        bi = best_idx[...]
        take = rowmin < bv
        best_val[...] = jnp.where(take, rowmin, bv)
        best_idx[...] = jnp.where(take, rowidx, bi)

    @pl.when(k_blk == nk - 1)
    def _emit():
        out_ref[...] = best_idx[...][:, None]


def _vq_argmin(z, codebook):
    n = z.shape[0]
    grid = (n // _BR, _K // _BK)
    return pl.pallas_call(
        _vq_kernel,
        grid=grid,
        in_specs=[
            pl.BlockSpec((_BR, _D), lambda r, k: (r, 0)),
            pl.BlockSpec((_BK, _D), lambda r, k: (k, 0)),
        ],
        out_specs=pl.BlockSpec((_BR, 1), lambda r, k: (r, 0)),
        out_shape=jax.ShapeDtypeStruct((n, 1), jnp.int32),
        scratch_shapes=[
            pltpu.VMEM((_BR,), jnp.float32),
            pltpu.VMEM((_BR,), jnp.int32),
        ],
    )(z, codebook)


def kernel(inputs, codebook):
    b, t, d = inputs.shape
    z = inputs.reshape(b * t, d)
    idx = _vq_argmin(z, codebook)
    return idx.reshape(b, t)


# speedup vs baseline: 3.5890x; 3.5890x over previous
import functools

import jax
import jax.numpy as jnp
from jax import lax
from jax.experimental import pallas as pl
from jax.experimental.pallas import tpu as pltpu
from jax.experimental.pallas import tpu_sc as plsc

_K = 8192
_D = 32
_NPAD = 800
_NW = 32
_RPW = 25
_BK = 512
_NB = _K // _BK
_NCAND = 2 * _NB
_BIG = 2**31 - 1


def _cand_kernel(z_ref, ct_ref, i1_ref, i2_ref):
    k_blk = pl.program_id(0)
    z = z_ref[...]
    ct = ct_ref[...]

    cn = jnp.sum(ct * ct, axis=0)
    dot = lax.dot_general(
        z, ct, (((1,), (0,)), ((), ())),
        preferred_element_type=jnp.float32,
        precision=lax.Precision.HIGHEST)
    s = cn[None, :] - (dot + dot)

    idx = lax.broadcasted_iota(jnp.int32, s.shape, 1) + k_blk * _BK
    m1 = jnp.min(s, axis=1)
    i1 = jnp.min(jnp.where(s == m1[:, None], idx, jnp.int32(_BIG)), axis=1)
    s2 = jnp.where(idx == i1[:, None], jnp.float32(jnp.inf), s)
    m2 = jnp.min(s2, axis=1)
    i2 = jnp.min(jnp.where(s2 == m2[:, None], idx, jnp.int32(_BIG)), axis=1)

    i1_ref[...] = i1[None, None, :]
    i2_ref[...] = i2[None, None, :]


def _candidates(z, ct):
    return pl.pallas_call(
        _cand_kernel,
        grid=(_NB,),
        in_specs=[
            pl.BlockSpec((_NPAD, _D), lambda k: (0, 0)),
            pl.BlockSpec((_D, _BK), lambda k: (0, k)),
        ],
        out_specs=[
            pl.BlockSpec((1, 1, _NPAD), lambda k: (k, 0, 0)),
            pl.BlockSpec((1, 1, _NPAD), lambda k: (k, 0, 0)),
        ],
        out_shape=[
            jax.ShapeDtypeStruct((_NB, 1, _NPAD), jnp.int32),
            jax.ShapeDtypeStruct((_NB, 1, _NPAD), jnp.int32),
        ],
    )(z, ct)


_GDN = lax.GatherDimensionNumbers(
    offset_dims=(), collapsed_slice_dims=(0,), start_index_map=(0,))


def _lperm(v, idx):
    return lax.gather(v, idx[:, None], _GDN, (1,),
                      mode=lax.GatherScatterMode.PROMISE_IN_BOUNDS)


def _sc_rescue_kernel(zw_hbm, i1_hbm, i2_hbm, cb_hbm, out_hbm,
                      zbuf, i1buf, i2buf, idxbuf, crows, ansbuf,
                      sem_in, sem_g):
    cid = lax.axis_index("c")
    sid = lax.axis_index("s")
    wid = sid * 2 + cid

    cp_z = pltpu.async_copy(zw_hbm.at[wid], zbuf, sem_in)
    cp_1 = pltpu.async_copy(i1_hbm.at[wid], i1buf, sem_in)
    cp_2 = pltpu.async_copy(i2_hbm.at[wid], i2buf, sem_in)
    cp_z.wait()
    cp_1.wait()
    cp_2.wait()

    for r in range(_RPW):
        idxbuf[pl.ds(r * _NCAND, _NB)] = i1buf[pl.ds(r * _NB, _NB)]
        idxbuf[pl.ds(r * _NCAND + _NB, _NB)] = i2buf[pl.ds(r * _NB, _NB)]

    gathers = []
    for r in range(_RPW):
        gathers.append(pltpu.async_copy(
            cb_hbm.at[idxbuf.at[pl.ds(r * _NCAND, _NCAND)]],
            crows.at[pl.ds(r * _NCAND, _NCAND)],
            sem_g))
    for g in gathers:
        g.wait()

    lanes = lax.iota(jnp.int32, 16)
    zeros = lanes * 0

    def row_body(r, acc):
        acc0, acc1 = acc
        z0 = zbuf[pl.ds(r * _D, 16)]
        z1 = zbuf[pl.ds(r * _D + 16, 16)]

        vals = []
        for g in range(2):
            val = jnp.full((16,), jnp.inf, jnp.float32)
            for j in range(_NB):
                row = r * _NCAND + g * _NB + j
                c0 = crows[row, pl.ds(0, 16)]
                c1 = crows[row, pl.ds(16, 16)]
                t0 = z0 - c0
                t1 = z1 - c1
                b = t0 * t0 + t1 * t1
                for st in (8, 4, 2, 1):
                    b = b + _lperm(b, (lanes + st) & 15)
                bs = _lperm(b, zeros)
                val = jnp.where(lanes == j, bs, val)
            vals.append(val)

        kv0 = i1buf[pl.ds(r * _NB, _NB)]
        kv1 = i2buf[pl.ds(r * _NB, _NB)]
        v0, v1 = vals
        take = (v1 < v0) | ((v1 == v0) & (kv1 < kv0))
        bv = jnp.where(take, v1, v0)
        bi = jnp.where(take, kv1, kv0)

        for st in (8, 4, 2, 1):
            perm = (lanes + st) & 15
            v2 = _lperm(bv, perm)
            i2v = _lperm(bi, perm)
            t2 = (v2 < bv) | ((v2 == bv) & (i2v < bi))
            bv = jnp.where(t2, v2, bv)
            bi = jnp.where(t2, i2v, bi)
        ans = _lperm(bi, zeros)

        acc0 = jnp.where(lanes == r, ans, acc0)
        acc1 = jnp.where(lanes == (r - 16), ans, acc1)
        return acc0, acc1

    zero = jnp.zeros((16,), jnp.int32)
    acc0, acc1 = lax.fori_loop(0, _RPW, row_body, (zero, zero))
    ansbuf[pl.ds(0, 16)] = acc0
    ansbuf[pl.ds(16, 16)] = acc1
    pltpu.sync_copy(ansbuf, out_hbm.at[wid])


def _sc_rescue(zw, i1w, i2w, codebook):
    mesh = plsc.VectorSubcoreMesh(core_axis_name="c", subcore_axis_name="s")
    kern = functools.partial(
        pl.kernel,
        out_type=jax.ShapeDtypeStruct((_NW, 32), jnp.int32),
        mesh=mesh,
        scratch_types=[
            pltpu.VMEM((_RPW * _D,), jnp.float32),
            pltpu.VMEM((_RPW * _NB,), jnp.int32),
            pltpu.VMEM((_RPW * _NB,), jnp.int32),
            pltpu.VMEM((_RPW * _NCAND,), jnp.int32),
            pltpu.VMEM((_RPW * _NCAND, 128), jnp.float32),
            pltpu.VMEM((32,), jnp.int32),
            pltpu.SemaphoreType.DMA,
            pltpu.SemaphoreType.DMA,
        ],
    )(_sc_rescue_kernel)
    return kern(zw, i1w, i2w, codebook)


def kernel(inputs, codebook):
    b, t, d = inputs.shape
    z = inputs.reshape(b * t, d)
    z = jnp.pad(z, ((0, _NPAD - b * t), (0, 0)))

    ct = codebook.T
    i1, i2 = _candidates(z, ct)

    i1w = i1[:, 0, :].T.reshape(_NW, _RPW * _NB)
    i2w = i2[:, 0, :].T.reshape(_NW, _RPW * _NB)
    zw = z.reshape(_NW, _RPW * _D)

    cb_pad = jnp.pad(codebook, ((0, 0), (0, 128 - _D)))

    out = _sc_rescue(zw, i1w, i2w, cb_pad)
    idx = out[:, :_RPW].reshape(_NW * _RPW)[: b * t]
    return idx.reshape(b, t)


# speedup vs baseline: 5.6585x; 1.5766x over previous
import functools

import jax
import jax.numpy as jnp
from jax import lax
from jax.experimental import pallas as pl
from jax.experimental.pallas import tpu as pltpu
from jax.experimental.pallas import tpu_sc as plsc

_K = 8192
_D = 32
_NPAD = 800
_NW = 32
_RPW = 25
_BK = 1024
_NB = _K // _BK
_NCAND = 2 * _NB
_BIG = 2**31 - 1


def _cand_kernel(z2_ref, ct_ref, i1_ref, i2_ref):
    k_blk = pl.program_id(0)
    z2 = z2_ref[...]
    ct = ct_ref[...]

    cn = jnp.sum(ct * ct, axis=0)
    ct_aug = jnp.concatenate([ct, cn[None, :]], axis=0)
    s = lax.dot_general(
        ct_aug, z2, (((0,), (1,)), ((), ())),
        preferred_element_type=jnp.float32,
        precision=lax.Precision.HIGHEST)

    idx = lax.broadcasted_iota(jnp.int32, s.shape, 0) + k_blk * _BK
    m1 = jnp.min(s, axis=0)
    i1 = jnp.min(jnp.where(s == m1[None, :], idx, jnp.int32(_BIG)), axis=0)
    s2 = jnp.where(idx == i1[None, :], jnp.float32(jnp.inf), s)
    m2 = jnp.min(s2, axis=0)
    i2 = jnp.min(jnp.where(s2 == m2[None, :], idx, jnp.int32(_BIG)), axis=0)

    i1_ref[...] = i1[None, None, :]
    i2_ref[...] = i2[None, None, :]


def _candidates(z2, ct):
    return pl.pallas_call(
        _cand_kernel,
        grid=(_NB,),
        in_specs=[
            pl.BlockSpec((_NPAD, _D + 1), lambda k: (0, 0)),
            pl.BlockSpec((_D, _BK), lambda k: (0, k)),
        ],
        out_specs=[
            pl.BlockSpec((1, 1, _NPAD), lambda k: (k, 0, 0)),
            pl.BlockSpec((1, 1, _NPAD), lambda k: (k, 0, 0)),
        ],
        out_shape=[
            jax.ShapeDtypeStruct((_NB, 1, _NPAD), jnp.int32),
            jax.ShapeDtypeStruct((_NB, 1, _NPAD), jnp.int32),
        ],
    )(z2, ct)


_GDN = lax.GatherDimensionNumbers(
    offset_dims=(), collapsed_slice_dims=(0,), start_index_map=(0,))


def _lperm(v, idx):
    return lax.gather(v, idx[:, None], _GDN, (1,),
                      mode=lax.GatherScatterMode.PROMISE_IN_BOUNDS)


def _sc_rescue_kernel(zw_hbm, cand_hbm, cb_hbm, out_hbm,
                      zbuf, candbuf, crows, ansbuf, sem_in, sem_g):
    cid = lax.axis_index("c")
    sid = lax.axis_index("s")
    wid = sid * 2 + cid

    cp_z = pltpu.async_copy(zw_hbm.at[wid], zbuf, sem_in)
    cp_c = pltpu.async_copy(cand_hbm.at[wid], candbuf, sem_in)
    cp_z.wait()
    cp_c.wait()

    gathers = []
    for g in range(5):
        gathers.append(pltpu.async_copy(
            cb_hbm.at[candbuf.at[pl.ds(g * 80, 80)]],
            crows.at[pl.ds(g * 80, 80)],
            sem_g))
    for g in gathers:
        g.wait()

    lanes = lax.iota(jnp.int32, 16)
    zeros = lanes * 0

    def row_body(r, acc):
        acc0, acc1 = acc
        z0 = zbuf[pl.ds(r * _D, 16)]
        z1 = zbuf[pl.ds(r * _D + 16, 16)]

        val = jnp.full((16,), jnp.inf, jnp.float32)
        for j in range(_NCAND):
            row = r * _NCAND + j
            c0 = crows[row, pl.ds(0, 16)]
            c1 = crows[row, pl.ds(16, 16)]
            t0 = z0 - c0
            t1 = z1 - c1
            b = t0 * t0 + t1 * t1
            for st in (8, 4, 2, 1):
                b = b + _lperm(b, (lanes + st) & 15)
            bs = _lperm(b, zeros)
            val = jnp.where(lanes == j, bs, val)

        bi = candbuf[pl.ds(r * _NCAND, _NCAND)]
        bv = val
        for st in (8, 4, 2, 1):
            perm = (lanes + st) & 15
            v2 = _lperm(bv, perm)
            i2v = _lperm(bi, perm)
            t2 = (v2 < bv) | ((v2 == bv) & (i2v < bi))
            bv = jnp.where(t2, v2, bv)
            bi = jnp.where(t2, i2v, bi)
        ans = _lperm(bi, zeros)

        acc0 = jnp.where(lanes == r, ans, acc0)
        acc1 = jnp.where(lanes == (r - 16), ans, acc1)
        return acc0, acc1

    zero = jnp.zeros((16,), jnp.int32)
    acc0, acc1 = lax.fori_loop(0, _RPW, row_body, (zero, zero))
    ansbuf[pl.ds(0, 16)] = acc0
    ansbuf[pl.ds(16, 16)] = acc1
    pltpu.sync_copy(ansbuf, out_hbm.at[wid])


def _sc_rescue(zw, candw, cb_pad):
    mesh = plsc.VectorSubcoreMesh(core_axis_name="c", subcore_axis_name="s")
    kern = functools.partial(
        pl.kernel,
        out_type=jax.ShapeDtypeStruct((_NW, 32), jnp.int32),
        mesh=mesh,
        scratch_types=[
            pltpu.VMEM((_RPW * _D,), jnp.float32),
            pltpu.VMEM((_RPW * _NCAND,), jnp.int32),
            pltpu.VMEM((_RPW * _NCAND, 128), jnp.float32),
            pltpu.VMEM((32,), jnp.int32),
            pltpu.SemaphoreType.DMA,
            pltpu.SemaphoreType.DMA,
        ],
    )(_sc_rescue_kernel)
    return kern(zw, candw, cb_pad)


def kernel(inputs, codebook):
    b, t, d = inputs.shape
    z = inputs.reshape(b * t, d)
    z = jnp.pad(z, ((0, _NPAD - b * t), (0, 0)))

    ct = codebook.T
    z2 = jnp.concatenate(
        [z * jnp.float32(-2), jnp.ones((_NPAD, 1), jnp.float32)], axis=1)
    i1, i2 = _candidates(z2, ct)

    cands = jnp.concatenate([i1[:, 0, :].T, i2[:, 0, :].T], axis=1)
    candw = cands.reshape(_NW, _RPW * _NCAND)
    zw = z.reshape(_NW, _RPW * _D)
    cb_pad = jnp.pad(codebook, ((0, 0), (0, 128 - _D)))

    out = _sc_rescue(zw, candw, cb_pad)
    idx = out[:, :_RPW].reshape(_NW * _RPW)[: b * t]
    return idx.reshape(b, t)
